# baseline (device time: 68886 ns/iter reference)
import jax
import jax.numpy as jnp
from jax import lax
from jax.experimental import pallas as pl
from jax.experimental.pallas import tpu as pltpu

N_DEV = 4
SQ = 512
D = 1024
HEADS = 8
DH = 128
SCALE = 0.08838834764831843

C16 = jnp.bfloat16


def kernel(x, Wq, Wo, Wk, Wv):
    def body(x_ref, wq_ref, wo_ref, wk_ref, wv_ref, out_ref,
             xg_ref, pin_ref, pout_ref, w16_ref, wo16_ref,
             x_send_sems, x_recv_sems, p_send_sems, p_recv_sems):
        my = lax.axis_index("i")
        peers = [lax.rem(my + k, N_DEV) for k in range(1, N_DEV)]

        barrier = pltpu.get_barrier_semaphore()
        for p in peers:
            pl.semaphore_signal(barrier, inc=1, device_id=(p,),
                                device_id_type=pl.DeviceIdType.MESH)
        pl.semaphore_wait(barrier, N_DEV - 1)

        xg_ref[pl.ds(my, 1)] = x_ref[...].astype(C16)

        def send_x(k):
            rdma = pltpu.make_async_remote_copy(
                src_ref=xg_ref.at[pl.ds(my, 1)],
                dst_ref=xg_ref.at[pl.ds(my, 1)],
                send_sem=x_send_sems.at[k - 1],
                recv_sem=x_recv_sems.at[3 - k],
                device_id=(peers[k - 1],),
                device_id_type=pl.DeviceIdType.MESH,
            )
            rdma.start()
            return rdma

        sends = [send_x(1), send_x(3)]

        w16_ref[:, pl.ds(0, D)] = wq_ref[...].astype(C16)
        w16_ref[:, pl.ds(D, D)] = wk_ref[...].astype(C16)
        w16_ref[:, pl.ds(2 * D, D)] = wv_ref[...].astype(C16)
        wo16_ref[...] = wo_ref[...].astype(C16)

        def attn_rows(qkv16, r0, rn):
            outs = []
            for hh in range(HEADS):
                q16 = qkv16[r0:r0 + rn, hh * DH:(hh + 1) * DH]
                k16 = qkv16[:, D + hh * DH:D + (hh + 1) * DH]
                v16 = qkv16[:, 2 * D + hh * DH:2 * D + (hh + 1) * DH]
                s = jnp.dot(q16, k16.T,
                            preferred_element_type=jnp.float32) * SCALE
                p16 = jnp.exp(s).astype(C16)
                lsum = jnp.sum(p16, axis=1, keepdims=True,
                               dtype=jnp.float32)
                o = jnp.dot(p16, v16, preferred_element_type=jnp.float32)
                outs.append((o / lsum).astype(C16))
            return jnp.concatenate(outs, axis=1)

        def qkv_proj(xb16):
            return jnp.dot(xb16, w16_ref[...],
                           preferred_element_type=jnp.float32).astype(C16)

        def attn_partial(xb16, out_dtype):
            res = jnp.dot(attn_rows(qkv_proj(xb16), 0, SQ), wo16_ref[...],
                          preferred_element_type=jnp.float32)
            return res if out_dtype == jnp.float32 else res.astype(out_dtype)

        out_ref[...] = attn_partial(xg_ref[pl.ds(my, 1)][0], jnp.float32)[None]

        sends[0].wait_send()
        sends[1].wait_send()
        sends = [send_x(2)]

        def recv_x(o):
            b = lax.rem(my + o, N_DEV)
            rdma = pltpu.make_async_remote_copy(
                src_ref=xg_ref.at[pl.ds(b, 1)],
                dst_ref=xg_ref.at[pl.ds(b, 1)],
                send_sem=x_send_sems.at[0],
                recv_sem=x_recv_sems.at[o - 1],
                device_id=(my,),
                device_id_type=pl.DeviceIdType.MESH,
            )
            rdma.wait_recv()
            return b

        HALF = SQ // 2
        for o in (1, 3, 2):
            b = recv_x(o)
            qkv16 = qkv_proj(xg_ref[pl.ds(b, 1)][0])
            for i in range(2):
                rows = pl.ds(i * HALF, HALF)
                pout_ref[o - 1, rows] = jnp.dot(
                    attn_rows(qkv16, i * HALF, HALF), wo16_ref[...],
                    preferred_element_type=jnp.float32).astype(C16)
                rdma = pltpu.make_async_remote_copy(
                    src_ref=pout_ref.at[o - 1, rows],
                    dst_ref=pin_ref.at[3 - o, rows],
                    send_sem=p_send_sems.at[2 * (o - 1) + i],
                    recv_sem=p_recv_sems.at[2 * (3 - o) + i],
                    device_id=(b,),
                    device_id_type=pl.DeviceIdType.MESH,
                )
                rdma.start()
                sends.append(rdma)

        for j in (2, 0, 1):
            for i in range(2):
                rdma = pltpu.make_async_remote_copy(
                    src_ref=pout_ref.at[j, pl.ds(i * HALF, HALF)],
                    dst_ref=pin_ref.at[j, pl.ds(i * HALF, HALF)],
                    send_sem=p_send_sems.at[0],
                    recv_sem=p_recv_sems.at[2 * j + i],
                    device_id=(my,),
                    device_id_type=pl.DeviceIdType.MESH,
                )
                rdma.wait_recv()
            out_ref[...] = out_ref[...] + pin_ref[j].astype(jnp.float32)[None]

        for rdma in sends:
            rdma.wait_send()

    return pl.pallas_call(
        body,
        out_shape=jax.ShapeDtypeStruct((1, SQ, D), jnp.float32),
        in_specs=[pl.BlockSpec(memory_space=pltpu.VMEM)] * 5,
        out_specs=pl.BlockSpec(memory_space=pltpu.VMEM),
        scratch_shapes=[
            pltpu.VMEM((N_DEV, SQ, D), C16),
            pltpu.VMEM((N_DEV - 1, SQ, D), C16),
            pltpu.VMEM((N_DEV - 1, SQ, D), C16),
            pltpu.VMEM((D, 3 * D), C16),
            pltpu.VMEM((D, D), C16),
            pltpu.SemaphoreType.DMA((N_DEV - 1,)),
            pltpu.SemaphoreType.DMA((N_DEV - 1,)),
            pltpu.SemaphoreType.DMA((2 * (N_DEV - 1),)),
            pltpu.SemaphoreType.DMA((2 * (N_DEV - 1),)),
        ],
        compiler_params=pltpu.CompilerParams(
            collective_id=0, vmem_limit_bytes=100 * 1024 * 1024
        ),
    )(x, Wq, Wo, Wk, Wv)


# device time: 66078 ns/iter; 1.0425x vs baseline; 1.0425x over previous
import jax
import jax.numpy as jnp
from jax import lax
from jax.experimental import pallas as pl
from jax.experimental.pallas import tpu as pltpu

N_DEV = 4
SQ = 512
D = 1024
HEADS = 8
DH = 128
SCALE = 0.08838834764831843

C16 = jnp.bfloat16


def kernel(x, Wq, Wo, Wk, Wv):
    def body(x_ref, wq_ref, wo_ref, wk_ref, wv_ref, out_ref,
             xg_ref, pin_ref, pout_ref, w16_ref, wo16_ref,
             x_send_sems, x_recv_sems, p_send_sems, p_recv_sems):
        my = lax.axis_index("i")
        peers = [lax.rem(my + k, N_DEV) for k in range(1, N_DEV)]

        barrier = pltpu.get_barrier_semaphore()
        for p in peers:
            pl.semaphore_signal(barrier, inc=1, device_id=(p,),
                                device_id_type=pl.DeviceIdType.MESH)
        pl.semaphore_wait(barrier, N_DEV - 1)

        xg_ref[pl.ds(my, 1)] = x_ref[...].astype(C16)

        def send_x(k):
            rdma = pltpu.make_async_remote_copy(
                src_ref=xg_ref.at[pl.ds(my, 1)],
                dst_ref=xg_ref.at[pl.ds(my, 1)],
                send_sem=x_send_sems.at[k - 1],
                recv_sem=x_recv_sems.at[3 - k],
                device_id=(peers[k - 1],),
                device_id_type=pl.DeviceIdType.MESH,
            )
            rdma.start()
            return rdma

        sends = [send_x(1), send_x(3)]

        w16_ref[:, pl.ds(0, D)] = wq_ref[...].astype(C16)
        w16_ref[:, pl.ds(D, D)] = wk_ref[...].astype(C16)
        w16_ref[:, pl.ds(2 * D, D)] = wv_ref[...].astype(C16)
        wo16_ref[...] = wo_ref[...].astype(C16)

        def attn_rows(qkv16, r0, rn):
            outs = []
            for hh in range(HEADS):
                q16 = qkv16[r0:r0 + rn, hh * DH:(hh + 1) * DH]
                k16 = qkv16[:, D + hh * DH:D + (hh + 1) * DH]
                v16 = qkv16[:, 2 * D + hh * DH:2 * D + (hh + 1) * DH]
                s = jnp.dot(q16, k16.T,
                            preferred_element_type=jnp.float32) * SCALE
                p16 = jnp.exp(s).astype(C16)
                lsum = jnp.sum(p16, axis=1, keepdims=True,
                               dtype=jnp.float32)
                o = jnp.dot(p16, v16, preferred_element_type=jnp.float32)
                outs.append((o / lsum).astype(C16))
            return jnp.concatenate(outs, axis=1)

        def qkv_proj(xb16):
            return jnp.dot(xb16, w16_ref[...],
                           preferred_element_type=jnp.float32).astype(C16)

        def attn_partial(xb16, out_dtype):
            res = jnp.dot(attn_rows(qkv_proj(xb16), 0, SQ), wo16_ref[...],
                          preferred_element_type=jnp.float32)
            return res if out_dtype == jnp.float32 else res.astype(out_dtype)

        out_ref[...] = attn_partial(xg_ref[pl.ds(my, 1)][0], jnp.float32)[None]

        sends[0].wait_send()
        sends[1].wait_send()
        sends = [send_x(2)]

        def recv_x(o):
            b = lax.rem(my + o, N_DEV)
            rdma = pltpu.make_async_remote_copy(
                src_ref=xg_ref.at[pl.ds(b, 1)],
                dst_ref=xg_ref.at[pl.ds(b, 1)],
                send_sem=x_send_sems.at[0],
                recv_sem=x_recv_sems.at[o - 1],
                device_id=(my,),
                device_id_type=pl.DeviceIdType.MESH,
            )
            rdma.wait_recv()
            return b

        HALF = SQ // 2
        for o in (1, 3, 2):
            b = recv_x(o)
            o_all = attn_rows(qkv_proj(xg_ref[pl.ds(b, 1)][0]), 0, SQ)
            for i in range(2):
                rows = pl.ds(i * HALF, HALF)
                pout_ref[o - 1, rows] = jnp.dot(
                    o_all[i * HALF:(i + 1) * HALF], wo16_ref[...],
                    preferred_element_type=jnp.float32).astype(C16)
                rdma = pltpu.make_async_remote_copy(
                    src_ref=pout_ref.at[o - 1, rows],
                    dst_ref=pin_ref.at[3 - o, rows],
                    send_sem=p_send_sems.at[2 * (o - 1) + i],
                    recv_sem=p_recv_sems.at[2 * (3 - o) + i],
                    device_id=(b,),
                    device_id_type=pl.DeviceIdType.MESH,
                )
                rdma.start()
                sends.append(rdma)

        for j in (2, 0, 1):
            for i in range(2):
                rdma = pltpu.make_async_remote_copy(
                    src_ref=pout_ref.at[j, pl.ds(i * HALF, HALF)],
                    dst_ref=pin_ref.at[j, pl.ds(i * HALF, HALF)],
                    send_sem=p_send_sems.at[0],
                    recv_sem=p_recv_sems.at[2 * j + i],
                    device_id=(my,),
                    device_id_type=pl.DeviceIdType.MESH,
                )
                rdma.wait_recv()
            out_ref[...] = out_ref[...] + pin_ref[j].astype(jnp.float32)[None]

        for rdma in sends:
            rdma.wait_send()

    return pl.pallas_call(
        body,
        out_shape=jax.ShapeDtypeStruct((1, SQ, D), jnp.float32),
        in_specs=[pl.BlockSpec(memory_space=pltpu.VMEM)] * 5,
        out_specs=pl.BlockSpec(memory_space=pltpu.VMEM),
        scratch_shapes=[
            pltpu.VMEM((N_DEV, SQ, D), C16),
            pltpu.VMEM((N_DEV - 1, SQ, D), C16),
            pltpu.VMEM((N_DEV - 1, SQ, D), C16),
            pltpu.VMEM((D, 3 * D), C16),
            pltpu.VMEM((D, D), C16),
            pltpu.SemaphoreType.DMA((N_DEV - 1,)),
            pltpu.SemaphoreType.DMA((N_DEV - 1,)),
            pltpu.SemaphoreType.DMA((2 * (N_DEV - 1),)),
            pltpu.SemaphoreType.DMA((2 * (N_DEV - 1),)),
        ],
        compiler_params=pltpu.CompilerParams(
            collective_id=0, vmem_limit_bytes=100 * 1024 * 1024
        ),
    )(x, Wq, Wo, Wk, Wv)


# device time: 66035 ns/iter; 1.0432x vs baseline; 1.0007x over previous
import jax
import jax.numpy as jnp
from jax import lax
from jax.experimental import pallas as pl
from jax.experimental.pallas import tpu as pltpu

N_DEV = 4
SQ = 512
D = 1024
HEADS = 8
DH = 128
SCALE = 0.08838834764831843

C16 = jnp.bfloat16


def kernel(x, Wq, Wo, Wk, Wv):
    def body(x_ref, wq_ref, wo_ref, wk_ref, wv_ref, out_ref,
             xg_ref, pin_ref, pout_ref, w16_ref, wo16_ref,
             x_send_sems, x_recv_sems, p_send_sems, p_recv_sems):
        my = lax.axis_index("i")
        peers = [lax.rem(my + k, N_DEV) for k in range(1, N_DEV)]

        barrier = pltpu.get_barrier_semaphore()
        for p in peers:
            pl.semaphore_signal(barrier, inc=1, device_id=(p,),
                                device_id_type=pl.DeviceIdType.MESH)
        pl.semaphore_wait(barrier, N_DEV - 1)

        xg_ref[pl.ds(my, 1)] = x_ref[...].astype(C16)

        def send_x(k):
            rdma = pltpu.make_async_remote_copy(
                src_ref=xg_ref.at[pl.ds(my, 1)],
                dst_ref=xg_ref.at[pl.ds(my, 1)],
                send_sem=x_send_sems.at[k - 1],
                recv_sem=x_recv_sems.at[3 - k],
                device_id=(peers[k - 1],),
                device_id_type=pl.DeviceIdType.MESH,
            )
            rdma.start()
            return rdma

        sends = [send_x(1), send_x(3)]

        w16_ref[:, pl.ds(0, D)] = wq_ref[...].astype(C16)
        w16_ref[:, pl.ds(D, D)] = wk_ref[...].astype(C16)
        w16_ref[:, pl.ds(2 * D, D)] = wv_ref[...].astype(C16)
        wo16_ref[...] = wo_ref[...].astype(C16)

        def attn_rows(qkv16, r0, rn):
            qq, kk, vv = qkv16
            outs = []
            for hh in range(HEADS):
                q16 = qq[r0:r0 + rn, hh * DH:(hh + 1) * DH]
                k16 = kk[:, hh * DH:(hh + 1) * DH]
                v16 = vv[:, hh * DH:(hh + 1) * DH]
                s = jnp.dot(q16, k16.T,
                            preferred_element_type=jnp.float32) * SCALE
                p16 = jnp.exp(s).astype(C16)
                lsum = jnp.sum(p16, axis=1, keepdims=True,
                               dtype=jnp.float32)
                o = jnp.dot(p16, v16, preferred_element_type=jnp.float32)
                outs.append((o / lsum).astype(C16))
            return jnp.concatenate(outs, axis=1)

        def qkv_proj(xb16):
            return tuple(
                jnp.dot(xb16, w16_ref[:, i * D:(i + 1) * D],
                        preferred_element_type=jnp.float32).astype(C16)
                for i in range(3)
            )

        def attn_partial(xb16, out_dtype):
            res = jnp.dot(attn_rows(qkv_proj(xb16), 0, SQ), wo16_ref[...],
                          preferred_element_type=jnp.float32)
            return res if out_dtype == jnp.float32 else res.astype(out_dtype)

        out_ref[...] = attn_partial(xg_ref[pl.ds(my, 1)][0], jnp.float32)[None]

        sends[0].wait_send()
        sends[1].wait_send()
        sends = [send_x(2)]

        def recv_x(o):
            b = lax.rem(my + o, N_DEV)
            rdma = pltpu.make_async_remote_copy(
                src_ref=xg_ref.at[pl.ds(b, 1)],
                dst_ref=xg_ref.at[pl.ds(b, 1)],
                send_sem=x_send_sems.at[0],
                recv_sem=x_recv_sems.at[o - 1],
                device_id=(my,),
                device_id_type=pl.DeviceIdType.MESH,
            )
            rdma.wait_recv()
            return b

        HALF = SQ // 2
        for o in (1, 3, 2):
            b = recv_x(o)
            o_all = attn_rows(qkv_proj(xg_ref[pl.ds(b, 1)][0]), 0, SQ)
            for i in range(2):
                rows = pl.ds(i * HALF, HALF)
                pout_ref[o - 1, rows] = jnp.dot(
                    o_all[i * HALF:(i + 1) * HALF], wo16_ref[...],
                    preferred_element_type=jnp.float32).astype(C16)
                rdma = pltpu.make_async_remote_copy(
                    src_ref=pout_ref.at[o - 1, rows],
                    dst_ref=pin_ref.at[3 - o, rows],
                    send_sem=p_send_sems.at[2 * (o - 1) + i],
                    recv_sem=p_recv_sems.at[2 * (3 - o) + i],
                    device_id=(b,),
                    device_id_type=pl.DeviceIdType.MESH,
                )
                rdma.start()
                sends.append(rdma)

        for j in (2, 0, 1):
            for i in range(2):
                rdma = pltpu.make_async_remote_copy(
                    src_ref=pout_ref.at[j, pl.ds(i * HALF, HALF)],
                    dst_ref=pin_ref.at[j, pl.ds(i * HALF, HALF)],
                    send_sem=p_send_sems.at[0],
                    recv_sem=p_recv_sems.at[2 * j + i],
                    device_id=(my,),
                    device_id_type=pl.DeviceIdType.MESH,
                )
                rdma.wait_recv()
            out_ref[...] = out_ref[...] + pin_ref[j].astype(jnp.float32)[None]

        for rdma in sends:
            rdma.wait_send()

    return pl.pallas_call(
        body,
        out_shape=jax.ShapeDtypeStruct((1, SQ, D), jnp.float32),
        in_specs=[pl.BlockSpec(memory_space=pltpu.VMEM)] * 5,
        out_specs=pl.BlockSpec(memory_space=pltpu.VMEM),
        scratch_shapes=[
            pltpu.VMEM((N_DEV, SQ, D), C16),
            pltpu.VMEM((N_DEV - 1, SQ, D), C16),
            pltpu.VMEM((N_DEV - 1, SQ, D), C16),
            pltpu.VMEM((D, 3 * D), C16),
            pltpu.VMEM((D, D), C16),
            pltpu.SemaphoreType.DMA((N_DEV - 1,)),
            pltpu.SemaphoreType.DMA((N_DEV - 1,)),
            pltpu.SemaphoreType.DMA((2 * (N_DEV - 1),)),
            pltpu.SemaphoreType.DMA((2 * (N_DEV - 1),)),
        ],
        compiler_params=pltpu.CompilerParams(
            collective_id=0, vmem_limit_bytes=100 * 1024 * 1024
        ),
    )(x, Wq, Wo, Wk, Wv)
